# baseline (speedup 1.0000x reference)
import functools

import jax
import jax.numpy as jnp
from jax import lax
from jax.experimental import pallas as pl
from jax.experimental.pallas import tpu as pltpu
from jax.experimental.pallas import tpu_sc as plsc

N = 10000
F = 128
NP = 10240
E = 320000
NC = 2
NS = 16
NW = NC * NS
C = 128
K = 80
NCH = NW * K
EPAD = NCH * C
TRASH = 10200
RPT = NP // NS

KS = 32
NST0 = 5
NST1 = 0
NCH0 = NS * NST0 * KS

_mesh = plsc.VectorSubcoreMesh(core_axis_name="c", subcore_axis_name="s")


@functools.partial(
    pl.kernel,
    out_type=jax.ShapeDtypeStruct((NC, NP, 16), jnp.float32),
    mesh=_mesh,
    scratch_types=[
        pltpu.VMEM_SHARED((NP, 16), jnp.float32),
        pltpu.VMEM((K, C), jnp.int32),
        pltpu.VMEM((C, 16), jnp.float32),
    ],
)
def _deg_kernel(col_hbm, out_hbm, dacc, cidx, ones):
    c = lax.axis_index("c")
    s = lax.axis_index("s")
    wid = s * NC + c
    base = wid * K

    @pl.loop(0, C)
    def _fill(i):
        ones[i, :] = jnp.ones((16,), jnp.float32)

    @pl.loop(0, RPT // C)
    def _init(k):
        pltpu.sync_copy(ones, dacc.at[pl.ds(s * RPT + k * C, C)])

    pltpu.sync_copy(col_hbm.at[pl.ds(base, K)], cidx)
    plsc.subcore_barrier()

    @pl.loop(0, K)
    def _scat(j):
        pltpu.sync_copy(ones, dacc.at[cidx.at[j]], add=True)

    plsc.subcore_barrier()
    pltpu.sync_copy(dacc.at[pl.ds(s * RPT, RPT)],
                    out_hbm.at[c, pl.ds(s * RPT, RPT)])


@functools.partial(
    pl.kernel,
    out_type=jax.ShapeDtypeStruct((NC, NP, F), jnp.float32),
    mesh=_mesh,
    scratch_types=[
        pltpu.VMEM_SHARED((NP, F), jnp.float32),
        pltpu.VMEM((KS, C), jnp.int32),
        pltpu.VMEM((KS, C), jnp.int32),
        pltpu.VMEM((C, F), jnp.float32),
        pltpu.VMEM((C, F), jnp.float32),
        pltpu.SemaphoreType.DMA,
        pltpu.SemaphoreType.DMA,
    ],
)
def _scatter_kernel(u_hbm, row_hbm, col_hbm, out_hbm, acc, ridx, cidx,
                    rows0, rows1, gsem0, gsem1):
    c = lax.axis_index("c")
    s = lax.axis_index("s")

    pltpu.sync_copy(u_hbm.at[pl.ds(s * RPT, RPT)], acc.at[pl.ds(s * RPT, RPT)])
    plsc.subcore_barrier()

    nst = jnp.where(c == 0, NST0, NST1)
    base = jnp.where(c == 0, s * (NST0 * KS), NCH0 + s * (NST1 * KS))

    @pl.loop(0, nst)
    def _stage(h):
        ch0 = base + h * KS
        pltpu.sync_copy(row_hbm.at[pl.ds(ch0, KS)], ridx)
        pltpu.sync_copy(col_hbm.at[pl.ds(ch0, KS)], cidx)
        pltpu.async_copy(u_hbm.at[ridx.at[0]], rows0, gsem0)

        @pl.loop(0, KS // 2)
        def _edge_chunk(i):
            j = i * 2
            pltpu.async_copy(u_hbm.at[ridx.at[j + 1]], rows1, gsem1)
            pltpu.make_async_copy(u_hbm.at[ridx.at[j]], rows0, gsem0).wait()
            pltpu.sync_copy(rows0, acc.at[cidx.at[j]], add=True)

            @pl.when(j + 2 < KS)
            def _():
                pltpu.async_copy(u_hbm.at[ridx.at[j + 2]], rows0, gsem0)

            pltpu.make_async_copy(u_hbm.at[ridx.at[j + 1]], rows1, gsem1).wait()
            pltpu.sync_copy(rows1, acc.at[cidx.at[j + 1]], add=True)

    plsc.subcore_barrier()
    pltpu.sync_copy(acc.at[pl.ds(s * RPT, RPT)],
                    out_hbm.at[c, pl.ds(s * RPT, RPT)])


_R = 1024


def _mm_scale_body(x_ref, w_ref, dinv_ref, o_ref):
    o_ref[...] = jnp.dot(x_ref[...], w_ref[...],
                         preferred_element_type=jnp.float32) * dinv_ref[...]


def _combine_mm_body(p_ref, dinv_ref, b_ref, w_ref, o_ref):
    y = (p_ref[0] + p_ref[1]) * dinv_ref[...] + b_ref[0:1, :]
    h = jnp.maximum(y, 0.0)
    o_ref[...] = jnp.dot(h, w_ref[...],
                         preferred_element_type=jnp.float32) * dinv_ref[...]


def _combine_lsm_body(p_ref, dinv_ref, b_ref, o_ref):
    y = (p_ref[0] + p_ref[1]) * dinv_ref[...] + b_ref[0:1, :]
    m = jnp.max(y, axis=1, keepdims=True)
    sh = y - m
    o_ref[...] = sh - jnp.log(jnp.sum(jnp.exp(sh), axis=1, keepdims=True))


_row_spec = pl.BlockSpec((_R, F), lambda i: (i, 0))
_w_spec = pl.BlockSpec((F, F), lambda i: (0, 0))
_b_spec = pl.BlockSpec((8, F), lambda i: (0, 0))
_p_spec = pl.BlockSpec((NC, _R, F), lambda i: (0, i, 0))
_out_t = jax.ShapeDtypeStruct((NP, F), jnp.float32)
_grid = (NP // _R,)

_mm_scale = pl.pallas_call(
    _mm_scale_body, grid=_grid, out_shape=_out_t,
    in_specs=[_row_spec, _w_spec, _row_spec], out_specs=_row_spec)

_combine_mm = pl.pallas_call(
    _combine_mm_body, grid=_grid, out_shape=_out_t,
    in_specs=[_p_spec, _row_spec, _b_spec, _w_spec], out_specs=_row_spec)

_combine_lsm = pl.pallas_call(
    _combine_lsm_body, grid=_grid, out_shape=_out_t,
    in_specs=[_p_spec, _row_spec, _b_spec], out_specs=_row_spec)


def kernel(x, edge_index, W1, b1, W2, b2):
    row = edge_index[0]
    col = edge_index[1]
    pad = EPAD - E
    row_r = jnp.concatenate(
        [row, jnp.zeros((pad,), jnp.int32)]).reshape(NCH, C)
    col_r = jnp.concatenate(
        [col, jnp.full((pad,), TRASH, jnp.int32)]).reshape(NCH, C)
    x_p = jnp.pad(x, ((0, NP - N), (0, 0)))

    deg_p = _deg_kernel(col_r)
    deg = deg_p[0, :, 0] + deg_p[1, :, 0]
    dinv_b = jnp.broadcast_to(lax.rsqrt(deg)[:, None], (NP, F))

    b1_b = jnp.broadcast_to(b1[None, :], (8, F))
    b2_b = jnp.broadcast_to(b2[None, :], (8, F))

    u1 = _mm_scale(x_p, W1, dinv_b)
    p1 = _scatter_kernel(u1, row_r, col_r)
    u2 = _combine_mm(p1, dinv_b, b1_b, W2)
    p2 = _scatter_kernel(u2, row_r, col_r)
    out = _combine_lsm(p2, dinv_b, b2_b)
    return out[:N]


# speedup vs baseline: 3.4997x; 3.4997x over previous
import functools

import jax
import jax.numpy as jnp
from jax import lax
from jax.experimental import pallas as pl
from jax.experimental.pallas import tpu as pltpu
from jax.experimental.pallas import tpu_sc as plsc

N = 10000
F = 128
NP = 10240
E = 320000
NC = 2
NS = 16
NW = NC * NS
C = 128
K = 80
NCH = NW * K
EPAD = NCH * C
RPT = NP // NS

KS = 40
NST0 = 2
NST1 = 2
NCH0 = NS * NST0 * KS

_mesh = plsc.VectorSubcoreMesh(core_axis_name="c", subcore_axis_name="s")


@functools.partial(
    pl.kernel,
    out_type=jax.ShapeDtypeStruct((NC, NP, 16), jnp.float32),
    mesh=_mesh,
    scratch_types=[
        pltpu.VMEM_SHARED((NP, 16), jnp.float32),
        pltpu.VMEM((K, C), jnp.int32),
        pltpu.VMEM((C, 16), jnp.float32),
    ],
)
def _deg_kernel(col_hbm, out_hbm, dacc, cidx, ones):
    c = lax.axis_index("c")
    s = lax.axis_index("s")
    wid = s * NC + c
    base = wid * K

    @pl.loop(0, C)
    def _fill(i):
        ones[i, :] = jnp.ones((16,), jnp.float32)

    @pl.loop(0, RPT // C)
    def _init(k):
        pltpu.sync_copy(ones, dacc.at[pl.ds(s * RPT + k * C, C)])

    pltpu.sync_copy(col_hbm.at[pl.ds(base, K)], cidx)
    plsc.subcore_barrier()

    @pl.loop(0, K)
    def _scat(j):
        pltpu.sync_copy(ones, dacc.at[cidx.at[j]], add=True)

    plsc.subcore_barrier()
    pltpu.sync_copy(dacc.at[pl.ds(s * RPT, RPT)],
                    out_hbm.at[c, pl.ds(s * RPT, RPT)])


@functools.partial(
    pl.kernel,
    out_type=jax.ShapeDtypeStruct((NC, NP, F), jnp.float32),
    mesh=_mesh,
    scratch_types=[
        pltpu.VMEM_SHARED((NP, F), jnp.float32),
        pltpu.VMEM((KS, C), jnp.int32),
        pltpu.VMEM((KS, C), jnp.int32),
        pltpu.VMEM((C, F), jnp.float32),
        pltpu.VMEM((C, F), jnp.float32),
        pltpu.SemaphoreType.DMA,
        pltpu.SemaphoreType.DMA,
    ],
)
def _scatter_kernel(u_hbm, row_hbm, col_hbm, out_hbm, acc, ridx, cidx,
                    rows0, rows1, gsem0, gsem1):
    c = lax.axis_index("c")
    s = lax.axis_index("s")

    pltpu.sync_copy(u_hbm.at[pl.ds(s * RPT, RPT)], acc.at[pl.ds(s * RPT, RPT)])
    plsc.subcore_barrier()

    nst = jnp.where(c == 0, NST0, NST1)
    base = jnp.where(c == 0, s * (NST0 * KS), NCH0 + s * (NST1 * KS))

    @pl.loop(0, nst)
    def _stage(h):
        ch0 = base + h * KS
        pltpu.sync_copy(row_hbm.at[pl.ds(ch0, KS)], ridx)
        pltpu.sync_copy(col_hbm.at[pl.ds(ch0, KS)], cidx)
        pltpu.async_copy(u_hbm.at[ridx.at[0]], rows0, gsem0)

        @pl.loop(0, KS // 2)
        def _edge_chunk(i):
            j = i * 2
            pltpu.async_copy(u_hbm.at[ridx.at[j + 1]], rows1, gsem1)
            pltpu.make_async_copy(u_hbm.at[ridx.at[j]], rows0, gsem0).wait()
            pltpu.sync_copy(rows0, acc.at[cidx.at[j]], add=True)

            @pl.when(j + 2 < KS)
            def _():
                pltpu.async_copy(u_hbm.at[ridx.at[j + 2]], rows0, gsem0)

            pltpu.make_async_copy(u_hbm.at[ridx.at[j + 1]], rows1, gsem1).wait()
            pltpu.sync_copy(rows1, acc.at[cidx.at[j + 1]], add=True)

    plsc.subcore_barrier()
    pltpu.sync_copy(acc.at[pl.ds(s * RPT, RPT)],
                    out_hbm.at[c, pl.ds(s * RPT, RPT)])


_R = 1024


def _mm_scale_body(x_ref, w_ref, dinv_ref, o_ref):
    o_ref[...] = jnp.dot(x_ref[...], w_ref[...],
                         preferred_element_type=jnp.float32) * dinv_ref[...]


def _combine_mm_body(p_ref, dinv_ref, b_ref, w_ref, o_ref):
    y = (p_ref[0] + p_ref[1]) * dinv_ref[...] + b_ref[0:1, :]
    h = jnp.maximum(y, 0.0)
    o_ref[...] = jnp.dot(h, w_ref[...],
                         preferred_element_type=jnp.float32) * dinv_ref[...]


def _combine_lsm_body(p_ref, dinv_ref, b_ref, o_ref):
    y = (p_ref[0] + p_ref[1]) * dinv_ref[...] + b_ref[0:1, :]
    m = jnp.max(y, axis=1, keepdims=True)
    sh = y - m
    o_ref[...] = sh - jnp.log(jnp.sum(jnp.exp(sh), axis=1, keepdims=True))


_row_spec = pl.BlockSpec((_R, F), lambda i: (i, 0))
_w_spec = pl.BlockSpec((F, F), lambda i: (0, 0))
_b_spec = pl.BlockSpec((8, F), lambda i: (0, 0))
_p_spec = pl.BlockSpec((NC, _R, F), lambda i: (0, i, 0))
_out_t = jax.ShapeDtypeStruct((NP, F), jnp.float32)
_grid = (NP // _R,)

_mm_scale = pl.pallas_call(
    _mm_scale_body, grid=_grid, out_shape=_out_t,
    in_specs=[_row_spec, _w_spec, _row_spec], out_specs=_row_spec)

_combine_mm = pl.pallas_call(
    _combine_mm_body, grid=_grid, out_shape=_out_t,
    in_specs=[_p_spec, _row_spec, _b_spec, _w_spec], out_specs=_row_spec)

_combine_lsm = pl.pallas_call(
    _combine_lsm_body, grid=_grid, out_shape=_out_t,
    in_specs=[_p_spec, _row_spec, _b_spec], out_specs=_row_spec)


def kernel(x, edge_index, W1, b1, W2, b2):
    row = edge_index[0]
    col = edge_index[1]
    pad = EPAD - E
    pad_ar = jnp.arange(pad, dtype=jnp.int32)
    row_r = jnp.concatenate(
        [row, pad_ar % N]).reshape(NCH, C)
    col_r = jnp.concatenate(
        [col, N + pad_ar % (NP - N)]).reshape(NCH, C)
    x_p = jnp.pad(x, ((0, NP - N), (0, 0)))

    deg_p = _deg_kernel(col_r)
    deg = deg_p[0, :, 0] + deg_p[1, :, 0]
    dinv_b = jnp.broadcast_to(lax.rsqrt(deg)[:, None], (NP, F))

    b1_b = jnp.broadcast_to(b1[None, :], (8, F))
    b2_b = jnp.broadcast_to(b2[None, :], (8, F))

    u1 = _mm_scale(x_p, W1, dinv_b)
    p1 = _scatter_kernel(u1, row_r, col_r)
    u2 = _combine_mm(p1, dinv_b, b1_b, W2)
    p2 = _scatter_kernel(u2, row_r, col_r)
    out = _combine_lsm(p2, dinv_b, b2_b)
    return out[:N]


# speedup vs baseline: 3.5198x; 1.0057x over previous
import functools

import jax
import jax.numpy as jnp
from jax import lax
from jax.experimental import pallas as pl
from jax.experimental.pallas import tpu as pltpu
from jax.experimental.pallas import tpu_sc as plsc

N = 10000
F = 128
NP = 10240
E = 320000
NC = 2
NS = 16
NW = NC * NS
C = 128
K = 80
NCH = NW * K
EPAD = NCH * C
RPT = NP // NS

KS = 40
NST0 = 2
NST1 = 2
NCH0 = NS * NST0 * KS

_mesh = plsc.VectorSubcoreMesh(core_axis_name="c", subcore_axis_name="s")


@functools.partial(
    pl.kernel,
    out_type=jax.ShapeDtypeStruct((NC, NP, 16), jnp.float32),
    mesh=_mesh,
    scratch_types=[
        pltpu.VMEM_SHARED((NP, 16), jnp.float32),
        pltpu.VMEM((K, C), jnp.int32),
        pltpu.VMEM((C, 16), jnp.float32),
    ],
)
def _deg_kernel(col_hbm, out_hbm, dacc, cidx, ones):
    c = lax.axis_index("c")
    s = lax.axis_index("s")
    wid = s * NC + c
    base = wid * K

    @pl.loop(0, C)
    def _fill(i):
        ones[i, :] = jnp.ones((16,), jnp.float32)

    @pl.loop(0, RPT // C)
    def _init(k):
        pltpu.sync_copy(ones, dacc.at[pl.ds(s * RPT + k * C, C)])

    pltpu.sync_copy(col_hbm.at[pl.ds(base, K)], cidx)
    plsc.subcore_barrier()

    @pl.loop(0, K)
    def _scat(j):
        pltpu.sync_copy(ones, dacc.at[cidx.at[j]], add=True)

    plsc.subcore_barrier()
    pltpu.sync_copy(dacc.at[pl.ds(s * RPT, RPT)],
                    out_hbm.at[c, pl.ds(s * RPT, RPT)])


@functools.partial(
    pl.kernel,
    out_type=jax.ShapeDtypeStruct((NC, NP, F), jnp.float32),
    mesh=_mesh,
    scratch_types=[
        pltpu.VMEM_SHARED((NP, F), jnp.float32),
        pltpu.VMEM((KS, C), jnp.int32),
        pltpu.VMEM((KS, C), jnp.int32),
        pltpu.VMEM((C, F), jnp.float32),
        pltpu.VMEM((C, F), jnp.float32),
        pltpu.SemaphoreType.DMA,
        pltpu.SemaphoreType.DMA,
    ],
)
def _scatter_kernel(u_hbm, row_hbm, col_hbm, out_hbm, acc, ridx, cidx,
                    rows0, rows1, gsem0, gsem1):
    c = lax.axis_index("c")
    s = lax.axis_index("s")

    pltpu.sync_copy(u_hbm.at[pl.ds(s * RPT, RPT)], acc.at[pl.ds(s * RPT, RPT)])
    plsc.subcore_barrier()

    nst = jnp.where(c == 0, NST0, NST1)
    base = jnp.where(c == 0, s * (NST0 * KS), NCH0 + s * (NST1 * KS))

    @pl.loop(0, nst)
    def _stage(h):
        ch0 = base + h * KS
        pltpu.sync_copy(row_hbm.at[pl.ds(ch0, KS)], ridx)
        pltpu.sync_copy(col_hbm.at[pl.ds(ch0, KS)], cidx)
        pltpu.async_copy(u_hbm.at[ridx.at[0]], rows0, gsem0)

        @pl.loop(0, KS // 2)
        def _edge_chunk(i):
            j = i * 2
            pltpu.async_copy(u_hbm.at[ridx.at[j + 1]], rows1, gsem1)
            pltpu.make_async_copy(u_hbm.at[ridx.at[j]], rows0, gsem0).wait()
            pltpu.sync_copy(rows0, acc.at[cidx.at[j]], add=True)

            @pl.when(j + 2 < KS)
            def _():
                pltpu.async_copy(u_hbm.at[ridx.at[j + 2]], rows0, gsem0)

            pltpu.make_async_copy(u_hbm.at[ridx.at[j + 1]], rows1, gsem1).wait()
            pltpu.sync_copy(rows1, acc.at[cidx.at[j + 1]], add=True)

    plsc.subcore_barrier()
    pltpu.sync_copy(acc.at[pl.ds(s * RPT, RPT)],
                    out_hbm.at[c, pl.ds(s * RPT, RPT)])


_R = 1024
_RO = 400


def _dinv(deg_ref):
    return lax.rsqrt((deg_ref[0] + deg_ref[1])[:, 0:1])


def _mm_scale_body(x_ref, w_ref, deg_ref, o_ref):
    o_ref[...] = jnp.dot(x_ref[...], w_ref[...],
                         preferred_element_type=jnp.float32) * _dinv(deg_ref)


def _combine_mm_body(p_ref, deg_ref, b_ref, w_ref, o_ref):
    dinv = _dinv(deg_ref)
    y = (p_ref[0] + p_ref[1]) * dinv + b_ref[0:1, :]
    h = jnp.maximum(y, 0.0)
    o_ref[...] = jnp.dot(h, w_ref[...],
                         preferred_element_type=jnp.float32) * dinv


def _combine_lsm_body(p_ref, deg_ref, b_ref, o_ref):
    y = (p_ref[0] + p_ref[1]) * _dinv(deg_ref) + b_ref[0:1, :]
    m = jnp.max(y, axis=1, keepdims=True)
    sh = y - m
    o_ref[...] = sh - jnp.log(jnp.sum(jnp.exp(sh), axis=1, keepdims=True))


_row_spec = pl.BlockSpec((_R, F), lambda i: (i, 0))
_w_spec = pl.BlockSpec((F, F), lambda i: (0, 0))
_b_spec = pl.BlockSpec((8, F), lambda i: (0, 0))
_p_spec = pl.BlockSpec((NC, _R, F), lambda i: (0, i, 0))
_deg_spec = pl.BlockSpec((NC, _R, 16), lambda i: (0, i, 0))
_out_t = jax.ShapeDtypeStruct((NP, F), jnp.float32)
_grid = (NP // _R,)

_mm_scale = pl.pallas_call(
    _mm_scale_body, grid=_grid, out_shape=_out_t,
    in_specs=[_row_spec, _w_spec, _deg_spec], out_specs=_row_spec)

_combine_mm = pl.pallas_call(
    _combine_mm_body, grid=_grid, out_shape=_out_t,
    in_specs=[_p_spec, _deg_spec, _b_spec, _w_spec], out_specs=_row_spec)

_combine_lsm = pl.pallas_call(
    _combine_lsm_body, grid=(N // _RO,),
    out_shape=jax.ShapeDtypeStruct((N, F), jnp.float32),
    in_specs=[pl.BlockSpec((NC, _RO, F), lambda i: (0, i, 0)),
              pl.BlockSpec((NC, _RO, 16), lambda i: (0, i, 0)),
              _b_spec],
    out_specs=pl.BlockSpec((_RO, F), lambda i: (i, 0)))


def kernel(x, edge_index, W1, b1, W2, b2):
    row = edge_index[0]
    col = edge_index[1]
    pad = EPAD - E
    pad_ar = jnp.arange(pad, dtype=jnp.int32)
    row_r = jnp.concatenate(
        [row, pad_ar % N]).reshape(NCH, C)
    col_r = jnp.concatenate(
        [col, N + pad_ar % (NP - N)]).reshape(NCH, C)
    x_p = jnp.pad(x, ((0, NP - N), (0, 0)))

    deg_p = _deg_kernel(col_r)

    b1_b = jnp.broadcast_to(b1[None, :], (8, F))
    b2_b = jnp.broadcast_to(b2[None, :], (8, F))

    u1 = _mm_scale(x_p, W1, deg_p)
    p1 = _scatter_kernel(u1, row_r, col_r)
    u2 = _combine_mm(p1, deg_p, b1_b, W2)
    p2 = _scatter_kernel(u2, row_r, col_r)
    return _combine_lsm(p2, deg_p, b2_b)


# speedup vs baseline: 3.5654x; 1.0130x over previous
import functools

import jax
import jax.numpy as jnp
from jax import lax
from jax.experimental import pallas as pl
from jax.experimental.pallas import tpu as pltpu
from jax.experimental.pallas import tpu_sc as plsc

N = 10000
F = 128
NP = 10240
E = 320000
NC = 2
NS = 16
NW = NC * NS
C = 128
K = 80
NCH = NW * K
EPAD = NCH * C
RPT = NP // NS

KS = 40
NST0 = 2
NST1 = 2
NCH0 = NS * NST0 * KS

_mesh = plsc.VectorSubcoreMesh(core_axis_name="c", subcore_axis_name="s")


@functools.partial(
    pl.kernel,
    out_type=jax.ShapeDtypeStruct((NC, NP, 16), jnp.float32),
    mesh=_mesh,
    scratch_types=[
        pltpu.VMEM_SHARED((NP, 16), jnp.float32),
        pltpu.VMEM((K, C), jnp.int32),
        pltpu.VMEM((C, 16), jnp.float32),
    ],
)
def _deg_kernel(col_hbm, out_hbm, dacc, cidx, ones):
    c = lax.axis_index("c")
    s = lax.axis_index("s")
    wid = s * NC + c
    base = wid * K

    @pl.loop(0, C)
    def _fill(i):
        ones[i, :] = jnp.ones((16,), jnp.float32)

    @pl.loop(0, RPT // C)
    def _init(k):
        pltpu.sync_copy(ones, dacc.at[pl.ds(s * RPT + k * C, C)])

    pltpu.sync_copy(col_hbm.at[pl.ds(base, K)], cidx)
    plsc.subcore_barrier()

    @pl.loop(0, K)
    def _scat(j):
        pltpu.sync_copy(ones, dacc.at[cidx.at[j]], add=True)

    plsc.subcore_barrier()
    pltpu.sync_copy(dacc.at[pl.ds(s * RPT, RPT)],
                    out_hbm.at[c, pl.ds(s * RPT, RPT)])


@functools.partial(
    pl.kernel,
    out_type=jax.ShapeDtypeStruct((NC, NP, F), jnp.float32),
    mesh=_mesh,
    scratch_types=[
        pltpu.VMEM_SHARED((NP, F), jnp.float32),
        pltpu.VMEM((KS, C), jnp.int32),
        pltpu.VMEM((KS, C), jnp.int32),
        pltpu.VMEM((C, F), jnp.float32),
        pltpu.VMEM((C, F), jnp.float32),
        pltpu.SemaphoreType.DMA,
        pltpu.SemaphoreType.DMA,
    ],
)
def _scatter_kernel(u_hbm, row_hbm, col_hbm, out_hbm, acc, ridx, cidx,
                    rows0, rows1, gsem0, gsem1):
    c = lax.axis_index("c")
    s = lax.axis_index("s")

    base = jnp.where(c == 0, s * (NST0 * KS), NCH0 + s * (NST1 * KS))

    pltpu.sync_copy(row_hbm.at[pl.ds(base, KS)], ridx)
    pltpu.sync_copy(col_hbm.at[pl.ds(base, KS)], cidx)
    pltpu.async_copy(u_hbm.at[ridx.at[0]], rows0, gsem0)
    pltpu.async_copy(u_hbm.at[ridx.at[1]], rows1, gsem1)

    pltpu.sync_copy(u_hbm.at[pl.ds(s * RPT, RPT)], acc.at[pl.ds(s * RPT, RPT)])
    plsc.subcore_barrier()

    for h in range(NST0):
        if h > 0:
            ch0 = base + h * KS
            pltpu.sync_copy(row_hbm.at[pl.ds(ch0, KS)], ridx)
            pltpu.sync_copy(col_hbm.at[pl.ds(ch0, KS)], cidx)
            pltpu.async_copy(u_hbm.at[ridx.at[0]], rows0, gsem0)
            pltpu.async_copy(u_hbm.at[ridx.at[1]], rows1, gsem1)

        @pl.loop(0, KS // 2)
        def _edge_chunk(i):
            j = i * 2
            pltpu.make_async_copy(u_hbm.at[ridx.at[j]], rows0, gsem0).wait()
            pltpu.sync_copy(rows0, acc.at[cidx.at[j]], add=True)

            @pl.when(j + 2 < KS)
            def _():
                pltpu.async_copy(u_hbm.at[ridx.at[j + 2]], rows0, gsem0)

            pltpu.make_async_copy(u_hbm.at[ridx.at[j + 1]], rows1, gsem1).wait()
            pltpu.sync_copy(rows1, acc.at[cidx.at[j + 1]], add=True)

            @pl.when(j + 3 < KS)
            def _():
                pltpu.async_copy(u_hbm.at[ridx.at[j + 3]], rows1, gsem1)

    plsc.subcore_barrier()
    pltpu.sync_copy(acc.at[pl.ds(s * RPT, RPT)],
                    out_hbm.at[c, pl.ds(s * RPT, RPT)])


_R = 1024
_RO = 400


def _dinv(deg_ref):
    return lax.rsqrt((deg_ref[0] + deg_ref[1])[:, 0:1])


def _mm_scale_body(x_ref, w_ref, deg_ref, o_ref):
    o_ref[...] = jnp.dot(x_ref[...], w_ref[...],
                         preferred_element_type=jnp.float32) * _dinv(deg_ref)


def _combine_mm_body(p_ref, deg_ref, b_ref, w_ref, o_ref):
    dinv = _dinv(deg_ref)
    y = (p_ref[0] + p_ref[1]) * dinv + b_ref[0:1, :]
    h = jnp.maximum(y, 0.0)
    o_ref[...] = jnp.dot(h, w_ref[...],
                         preferred_element_type=jnp.float32) * dinv


def _combine_lsm_body(p_ref, deg_ref, b_ref, o_ref):
    y = (p_ref[0] + p_ref[1]) * _dinv(deg_ref) + b_ref[0:1, :]
    m = jnp.max(y, axis=1, keepdims=True)
    sh = y - m
    o_ref[...] = sh - jnp.log(jnp.sum(jnp.exp(sh), axis=1, keepdims=True))


_row_spec = pl.BlockSpec((_R, F), lambda i: (i, 0))
_w_spec = pl.BlockSpec((F, F), lambda i: (0, 0))
_b_spec = pl.BlockSpec((8, F), lambda i: (0, 0))
_p_spec = pl.BlockSpec((NC, _R, F), lambda i: (0, i, 0))
_deg_spec = pl.BlockSpec((NC, _R, 16), lambda i: (0, i, 0))
_out_t = jax.ShapeDtypeStruct((NP, F), jnp.float32)
_grid = (NP // _R,)

_mm_scale = pl.pallas_call(
    _mm_scale_body, grid=_grid, out_shape=_out_t,
    in_specs=[_row_spec, _w_spec, _deg_spec], out_specs=_row_spec)

_combine_mm = pl.pallas_call(
    _combine_mm_body, grid=_grid, out_shape=_out_t,
    in_specs=[_p_spec, _deg_spec, _b_spec, _w_spec], out_specs=_row_spec)

_combine_lsm = pl.pallas_call(
    _combine_lsm_body, grid=(N // _RO,),
    out_shape=jax.ShapeDtypeStruct((N, F), jnp.float32),
    in_specs=[pl.BlockSpec((NC, _RO, F), lambda i: (0, i, 0)),
              pl.BlockSpec((NC, _RO, 16), lambda i: (0, i, 0)),
              _b_spec],
    out_specs=pl.BlockSpec((_RO, F), lambda i: (i, 0)))


def kernel(x, edge_index, W1, b1, W2, b2):
    row = edge_index[0]
    col = edge_index[1]
    pad = EPAD - E
    pad_ar = jnp.arange(pad, dtype=jnp.int32)
    row_r = jnp.concatenate(
        [row, pad_ar % N]).reshape(NCH, C)
    col_r = jnp.concatenate(
        [col, N + pad_ar % (NP - N)]).reshape(NCH, C)
    x_p = jnp.pad(x, ((0, NP - N), (0, 0)))

    deg_p = _deg_kernel(col_r)

    b1_b = jnp.broadcast_to(b1[None, :], (8, F))
    b2_b = jnp.broadcast_to(b2[None, :], (8, F))

    u1 = _mm_scale(x_p, W1, deg_p)
    p1 = _scatter_kernel(u1, row_r, col_r)
    u2 = _combine_mm(p1, deg_p, b1_b, W2)
    p2 = _scatter_kernel(u2, row_r, col_r)
    return _combine_lsm(p2, deg_p, b2_b)


# speedup vs baseline: 3.6541x; 1.0249x over previous
import functools

import jax
import jax.numpy as jnp
from jax import lax
from jax.experimental import pallas as pl
from jax.experimental.pallas import tpu as pltpu
from jax.experimental.pallas import tpu_sc as plsc

N = 10000
F = 128
NP = 10240
E = 320000
NC = 2
NS = 16
NW = NC * NS
C = 128
K = 80
NCH = NW * K
EPAD = NCH * C
RPT = NP // NS

KS = 40
NST = 2

_mesh = plsc.VectorSubcoreMesh(core_axis_name="c", subcore_axis_name="s")


@functools.partial(
    pl.kernel,
    out_type=jax.ShapeDtypeStruct((NC, NP, 16), jnp.float32),
    mesh=_mesh,
    scratch_types=[
        pltpu.VMEM_SHARED((NP, 16), jnp.float32),
        pltpu.VMEM((K, C), jnp.int32),
        pltpu.VMEM((C, 16), jnp.float32),
    ],
)
def _deg_kernel(col_hbm, out_hbm, dacc, cidx, ones):
    c = lax.axis_index("c")
    s = lax.axis_index("s")
    wid = s * NC + c
    base = wid * K

    @pl.loop(0, C)
    def _fill(i):
        ones[i, :] = jnp.ones((16,), jnp.float32)

    @pl.loop(0, RPT // C)
    def _init(k):
        pltpu.sync_copy(ones, dacc.at[pl.ds(s * RPT + k * C, C)])

    pltpu.sync_copy(col_hbm.at[pl.ds(base, K)], cidx)
    plsc.subcore_barrier()

    @pl.loop(0, K)
    def _scat(j):
        pltpu.sync_copy(ones, dacc.at[cidx.at[j]], add=True)

    plsc.subcore_barrier()
    pltpu.sync_copy(dacc.at[pl.ds(s * RPT, RPT)],
                    out_hbm.at[c, pl.ds(s * RPT, RPT)])


@functools.partial(
    pl.kernel,
    out_type=jax.ShapeDtypeStruct((NC, NP, F), jnp.float32),
    mesh=_mesh,
    scratch_types=[
        pltpu.VMEM_SHARED((NP, F), jnp.float32),
        pltpu.VMEM((KS, C), jnp.int32),
        pltpu.VMEM((KS, C), jnp.int32),
        pltpu.VMEM((C, F), jnp.float32),
        pltpu.VMEM((C, F), jnp.float32),
        pltpu.SemaphoreType.DMA,
        pltpu.SemaphoreType.DMA,
    ],
)
def _scatter_kernel(u_hbm, row_hbm, col_hbm, out_hbm, acc, ridx, cidx,
                    rows0, rows1, gsem0, gsem1):
    c = lax.axis_index("c")
    s = lax.axis_index("s")
    wid = s * NC + c
    base = wid * K

    pltpu.sync_copy(row_hbm.at[pl.ds(base, KS)], ridx)
    pltpu.sync_copy(col_hbm.at[pl.ds(base, KS)], cidx)
    pltpu.async_copy(u_hbm.at[ridx.at[0]], rows0, gsem0)
    pltpu.async_copy(u_hbm.at[ridx.at[1]], rows1, gsem1)

    pltpu.sync_copy(u_hbm.at[pl.ds(s * RPT, RPT)], acc.at[pl.ds(s * RPT, RPT)])
    plsc.subcore_barrier()

    for h in range(NST):
        if h > 0:
            ch0 = base + h * KS
            pltpu.sync_copy(row_hbm.at[pl.ds(ch0, KS)], ridx)
            pltpu.sync_copy(col_hbm.at[pl.ds(ch0, KS)], cidx)
            pltpu.async_copy(u_hbm.at[ridx.at[0]], rows0, gsem0)
            pltpu.async_copy(u_hbm.at[ridx.at[1]], rows1, gsem1)

        @pl.loop(0, KS // 2)
        def _edge_chunk(i):
            j = i * 2
            pltpu.make_async_copy(u_hbm.at[ridx.at[j]], rows0, gsem0).wait()
            pltpu.sync_copy(rows0, acc.at[cidx.at[j]], add=True)

            @pl.when(j + 2 < KS)
            def _():
                pltpu.async_copy(u_hbm.at[ridx.at[j + 2]], rows0, gsem0)

            pltpu.make_async_copy(u_hbm.at[ridx.at[j + 1]], rows1, gsem1).wait()
            pltpu.sync_copy(rows1, acc.at[cidx.at[j + 1]], add=True)

            @pl.when(j + 3 < KS)
            def _():
                pltpu.async_copy(u_hbm.at[ridx.at[j + 3]], rows1, gsem1)

    plsc.subcore_barrier()
    pltpu.sync_copy(acc.at[pl.ds(s * RPT, RPT)],
                    out_hbm.at[c, pl.ds(s * RPT, RPT)])


_R = 1024
_RO = 1000


def _dinv(deg_ref):
    return lax.rsqrt((deg_ref[0] + deg_ref[1])[:, 0:1])


def _mm_scale_body(x_ref, w_ref, deg_ref, o_ref):
    o_ref[...] = jnp.dot(x_ref[...], w_ref[...],
                         preferred_element_type=jnp.float32) * _dinv(deg_ref)


def _combine_mm_body(p_ref, deg_ref, b_ref, w_ref, o_ref):
    dinv = _dinv(deg_ref)
    y = (p_ref[0] + p_ref[1]) * dinv + b_ref[0:1, :]
    h = jnp.maximum(y, 0.0)
    o_ref[...] = jnp.dot(h, w_ref[...],
                         preferred_element_type=jnp.float32) * dinv


def _combine_lsm_body(p_ref, deg_ref, b_ref, o_ref):
    y = (p_ref[0] + p_ref[1]) * _dinv(deg_ref) + b_ref[0:1, :]
    m = jnp.max(y, axis=1, keepdims=True)
    sh = y - m
    o_ref[...] = sh - jnp.log(jnp.sum(jnp.exp(sh), axis=1, keepdims=True))


_row_spec = pl.BlockSpec((_R, F), lambda i: (i, 0))
_w_spec = pl.BlockSpec((F, F), lambda i: (0, 0))
_b_spec = pl.BlockSpec((8, F), lambda i: (0, 0))
_p_spec = pl.BlockSpec((NC, _R, F), lambda i: (0, i, 0))
_deg_spec = pl.BlockSpec((NC, _R, 16), lambda i: (0, i, 0))
_out_t = jax.ShapeDtypeStruct((NP, F), jnp.float32)
_grid = (NP // _R,)

_mm_scale = pl.pallas_call(
    _mm_scale_body, grid=_grid, out_shape=_out_t,
    in_specs=[_row_spec, _w_spec, _deg_spec], out_specs=_row_spec)

_combine_mm = pl.pallas_call(
    _combine_mm_body, grid=_grid, out_shape=_out_t,
    in_specs=[_p_spec, _deg_spec, _b_spec, _w_spec], out_specs=_row_spec)

_combine_lsm = pl.pallas_call(
    _combine_lsm_body, grid=(N // _RO,),
    out_shape=jax.ShapeDtypeStruct((N, F), jnp.float32),
    in_specs=[pl.BlockSpec((NC, _RO, F), lambda i: (0, i, 0)),
              pl.BlockSpec((NC, _RO, 16), lambda i: (0, i, 0)),
              _b_spec],
    out_specs=pl.BlockSpec((_RO, F), lambda i: (i, 0)))


def kernel(x, edge_index, W1, b1, W2, b2):
    row = edge_index[0]
    col = edge_index[1]
    pad = EPAD - E
    pad_ar = jnp.arange(pad, dtype=jnp.int32)
    row_r = jnp.concatenate(
        [row, pad_ar % N]).reshape(NCH, C)
    col_r = jnp.concatenate(
        [col, N + pad_ar % (NP - N)]).reshape(NCH, C)
    x_p = jnp.pad(x, ((0, NP - N), (0, 0)))

    deg_p = _deg_kernel(col_r)

    b1_b = jnp.broadcast_to(b1[None, :], (8, F))
    b2_b = jnp.broadcast_to(b2[None, :], (8, F))

    u1 = _mm_scale(x_p, W1, deg_p)
    p1 = _scatter_kernel(u1, row_r, col_r)
    u2 = _combine_mm(p1, deg_p, b1_b, W2)
    p2 = _scatter_kernel(u2, row_r, col_r)
    return _combine_lsm(p2, deg_p, b2_b)
